# baseline (speedup 1.0000x reference)
import functools

import jax
import jax.numpy as jnp
import numpy as np
from jax import lax
from jax.experimental import pallas as pl
from jax.experimental.pallas import tpu as pltpu
from jax.experimental.pallas import tpu_sc as plsc

_DIM_LEN = 768
_GROUP = 4

_B = 2
_SPLIT = 8
_R = _B * _DIM_LEN * _SPLIT
_D = 224 * 224 // _SPLIT
_GROWS = _GROUP * _SPLIT
_NG = _R // _GROWS
_NC = 2
_NS = 16
_NW = _NC * _NS
_GPW = _NG // _NW


def _src_group_rows() -> np.ndarray:
    with jax.ensure_compile_time_eval():
        perm = np.asarray(jax.random.permutation(jax.random.key(1), _DIM_LEN // _GROUP))
    rows = (np.arange(_B)[:, None] * _DIM_LEN + perm[None, :] * _GROUP).reshape(-1)
    return (rows * _SPLIT).astype(np.int32)


_SRC_GROUPS = _src_group_rows()


def _permute_rows(x2, src):
    mesh = plsc.VectorSubcoreMesh(core_axis_name="c", subcore_axis_name="s")

    @functools.partial(
        pl.kernel,
        mesh=mesh,
        out_type=jax.ShapeDtypeStruct((_R, _D), jnp.float32),
        compiler_params=pltpu.CompilerParams(needs_layout_passes=False),
        scratch_types=[
            pltpu.VMEM((_NG,), jnp.int32),
            pltpu.SemaphoreType.DMA,
        ],
    )
    def k(x_hbm, src_hbm, out_hbm, idx_v, sem):
        wid = lax.axis_index("s") * _NC + lax.axis_index("c")
        base = wid * _GPW
        pltpu.sync_copy(src_hbm, idx_v)
        lanes = lax.iota(jnp.int32, 16)

        def src_row(j):
            flat = base + j
            aligned = pl.multiple_of((flat // 16) * 16, 16)
            vec = idx_v[pl.ds(aligned, 16)]
            srow = jnp.max(jnp.where(lanes == flat - aligned, vec, 0))
            return pl.multiple_of(srow, _GROWS)

        rows = [src_row(j) for j in range(_GPW)]
        for j in range(_GPW):
            pltpu.async_copy(
                x_hbm.at[pl.ds(rows[j], _GROWS)],
                out_hbm.at[pl.ds((base + j) * _GROWS, _GROWS)],
                sem,
            )
        for j in range(_GPW):
            pltpu.make_async_copy(
                x_hbm.at[pl.ds(rows[j], _GROWS)],
                out_hbm.at[pl.ds((base + j) * _GROWS, _GROWS)],
                sem,
            ).wait()

    return k(x2, src)


def kernel(x):
    x2 = x.reshape(_R, _D)
    src = jnp.asarray(_SRC_GROUPS)
    out2 = _permute_rows(x2, src)
    return out2.reshape(x.shape)


# speedup vs baseline: 7.2371x; 7.2371x over previous
import functools

import jax
import jax.numpy as jnp
import numpy as np
from jax import lax
from jax.experimental import pallas as pl
from jax.experimental.pallas import tpu as pltpu
from jax.experimental.pallas import tpu_sc as plsc

_DIM_LEN = 768
_GROUP = 4

_B = 2
_SPLIT = 8
_R = _B * _DIM_LEN * _SPLIT
_D = 224 * 224 // _SPLIT
_CROWS = 8
_NCH = _R // _CROWS
_NC = 2
_NS = 16
_NW = _NC * _NS
_CPW = _NCH // _NW


def _src_chunk_rows() -> np.ndarray:
    with jax.ensure_compile_time_eval():
        perm = np.asarray(jax.random.permutation(jax.random.key(1), _DIM_LEN // _GROUP))
    grp = (np.arange(_B)[:, None] * _DIM_LEN + perm[None, :] * _GROUP).reshape(-1)
    rows = grp[:, None] * _SPLIT + np.arange(_GROUP)[None, :] * _CROWS
    return rows.reshape(-1).astype(np.int32)


_SRC_CHUNKS = _src_chunk_rows()


def _permute_rows(x2, src):
    mesh = plsc.VectorSubcoreMesh(core_axis_name="c", subcore_axis_name="s")

    @functools.partial(
        pl.kernel,
        mesh=mesh,
        out_type=jax.ShapeDtypeStruct((_R, _D), jnp.float32),
        compiler_params=pltpu.CompilerParams(needs_layout_passes=False),
        scratch_types=[
            pltpu.VMEM((_NCH,), jnp.int32),
            pltpu.VMEM((_CROWS, _D), jnp.float32),
            pltpu.VMEM((_CROWS, _D), jnp.float32),
            pltpu.SemaphoreType.DMA,
            pltpu.SemaphoreType.DMA,
        ],
    )
    def k(x_hbm, src_hbm, out_hbm, idx_v, buf0, buf1, gsem, wsem):
        wid = lax.axis_index("s") * _NC + lax.axis_index("c")
        base = wid * _CPW
        bufs = (buf0, buf1)
        pltpu.sync_copy(src_hbm, idx_v)
        lanes = lax.iota(jnp.int32, 16)

        def src_row(c):
            flat = base + c
            aligned = pl.multiple_of((flat // 16) * 16, 16)
            vec = idx_v[pl.ds(aligned, 16)]
            srow = jnp.max(jnp.where(lanes == flat - aligned, vec, 0))
            return pl.multiple_of(srow, _CROWS)

        rows = [None] * _CPW

        def start_gather(c):
            rows[c] = src_row(c)
            pltpu.async_copy(x_hbm.at[pl.ds(rows[c], _CROWS)], bufs[c % 2], gsem)

        def wait_gather(c):
            pltpu.make_async_copy(
                x_hbm.at[pl.ds(rows[c], _CROWS)], bufs[c % 2], gsem
            ).wait()

        def start_write(c):
            pltpu.async_copy(
                bufs[c % 2], out_hbm.at[pl.ds((base + c) * _CROWS, _CROWS)], wsem
            )

        def wait_write(c):
            pltpu.make_async_copy(
                bufs[c % 2], out_hbm.at[pl.ds((base + c) * _CROWS, _CROWS)], wsem
            ).wait()

        for c in range(_CPW):
            if c >= 2:
                wait_write(c - 2)
            start_gather(c)
            if c >= 1:
                wait_gather(c - 1)
                start_write(c - 1)
        wait_gather(_CPW - 1)
        start_write(_CPW - 1)
        wait_write(_CPW - 2)
        wait_write(_CPW - 1)

    return k(x2, src)


def kernel(x):
    x2 = x.reshape(_R, _D)
    src = jnp.asarray(_SRC_CHUNKS)
    out2 = _permute_rows(x2, src)
    return out2.reshape(x.shape)


# speedup vs baseline: 7.3080x; 1.0098x over previous
import functools

import jax
import jax.numpy as jnp
import numpy as np
from jax import lax
from jax.experimental import pallas as pl
from jax.experimental.pallas import tpu as pltpu
from jax.experimental.pallas import tpu_sc as plsc

_DIM_LEN = 768
_GROUP = 4

_B = 2
_SPLIT = 8
_R = _B * _DIM_LEN * _SPLIT
_D = 224 * 224 // _SPLIT
_CROWS = 8
_NCH = _R // _CROWS
_NC = 2
_NS = 16
_NW = _NC * _NS
_CPW = _NCH // _NW


def _src_chunk_rows() -> np.ndarray:
    with jax.ensure_compile_time_eval():
        perm = np.asarray(jax.random.permutation(jax.random.key(1), _DIM_LEN // _GROUP))
    grp = (np.arange(_B)[:, None] * _DIM_LEN + perm[None, :] * _GROUP).reshape(-1)
    rows = grp[:, None] * _SPLIT + np.arange(_GROUP)[None, :] * _CROWS
    return rows.reshape(-1).astype(np.int32)


_SRC_CHUNKS = _src_chunk_rows()


def _permute_rows(x2, src):
    mesh = plsc.VectorSubcoreMesh(core_axis_name="c", subcore_axis_name="s")

    @functools.partial(
        pl.kernel,
        mesh=mesh,
        out_type=jax.ShapeDtypeStruct((_R, _D), jnp.float32),
        compiler_params=pltpu.CompilerParams(needs_layout_passes=False),
        scratch_types=[
            pltpu.VMEM((_NCH,), jnp.int32),
            pltpu.VMEM_SHARED((_NS, 2, _CROWS, _D), jnp.float32),
            pltpu.SemaphoreType.DMA,
            pltpu.SemaphoreType.DMA,
        ],
    )
    def k(x_hbm, src_hbm, out_hbm, idx_v, sbuf, gsem, wsem):
        sid = lax.axis_index("s")
        wid = sid * _NC + lax.axis_index("c")
        base = wid * _CPW
        bufs = (sbuf.at[sid, 0], sbuf.at[sid, 1])
        pltpu.sync_copy(src_hbm, idx_v)
        lanes = lax.iota(jnp.int32, 16)

        def src_row(c):
            flat = base + c
            aligned = pl.multiple_of((flat // 16) * 16, 16)
            vec = idx_v[pl.ds(aligned, 16)]
            srow = jnp.max(jnp.where(lanes == flat - aligned, vec, 0))
            return pl.multiple_of(srow, _CROWS)

        rows = [None] * _CPW

        def start_gather(c):
            rows[c] = src_row(c)
            pltpu.async_copy(x_hbm.at[pl.ds(rows[c], _CROWS)], bufs[c % 2], gsem)

        def wait_gather(c):
            pltpu.make_async_copy(
                x_hbm.at[pl.ds(rows[c], _CROWS)], bufs[c % 2], gsem
            ).wait()

        def start_write(c):
            pltpu.async_copy(
                bufs[c % 2], out_hbm.at[pl.ds((base + c) * _CROWS, _CROWS)], wsem
            )

        def wait_write(c):
            pltpu.make_async_copy(
                bufs[c % 2], out_hbm.at[pl.ds((base + c) * _CROWS, _CROWS)], wsem
            ).wait()

        for c in range(_CPW):
            if c >= 2:
                wait_write(c - 2)
            start_gather(c)
            if c >= 1:
                wait_gather(c - 1)
                start_write(c - 1)
        wait_gather(_CPW - 1)
        start_write(_CPW - 1)
        wait_write(_CPW - 2)
        wait_write(_CPW - 1)

    return k(x2, src)


def kernel(x):
    x2 = x.reshape(_R, _D)
    src = jnp.asarray(_SRC_CHUNKS)
    out2 = _permute_rows(x2, src)
    return out2.reshape(x.shape)


# speedup vs baseline: 12.1129x; 1.6575x over previous
import functools

import jax
import jax.numpy as jnp
import numpy as np
from jax import lax
from jax.experimental import pallas as pl
from jax.experimental.pallas import tpu as pltpu
from jax.experimental.pallas import tpu_sc as plsc

_DIM_LEN = 768
_GROUP = 4

_B = 2
_H = 224
_W = 224
_NCH = _B * _DIM_LEN
_NC = 2
_NS = 16
_NW = _NC * _NS
_CPW = _NCH // _NW


def _src_channels() -> np.ndarray:
    with jax.ensure_compile_time_eval():
        perm = np.asarray(jax.random.permutation(jax.random.key(1), _DIM_LEN // _GROUP))
    chan = (perm[:, None] * _GROUP + np.arange(_GROUP)[None, :]).reshape(-1)
    return chan.astype(np.int32)


_SRC_CHANNELS = _src_channels()


def _permute_channels(x, src):
    mesh = plsc.VectorSubcoreMesh(core_axis_name="c", subcore_axis_name="s")

    @functools.partial(
        pl.kernel,
        mesh=mesh,
        out_type=jax.ShapeDtypeStruct((_B, _DIM_LEN, _H, _W), jnp.float32),
        compiler_params=pltpu.CompilerParams(needs_layout_passes=False),
        scratch_types=[
            pltpu.VMEM((_DIM_LEN,), jnp.int32),
            pltpu.VMEM((_H, _W), jnp.float32),
            pltpu.VMEM((_H, _W), jnp.float32),
            pltpu.SemaphoreType.DMA,
            pltpu.SemaphoreType.DMA,
        ],
    )
    def k(x_hbm, src_hbm, out_hbm, idx_v, buf0, buf1, gsem, wsem):
        wid = lax.axis_index("s") * _NC + lax.axis_index("c")
        base = wid * _CPW
        bufs = (buf0, buf1)
        pltpu.sync_copy(src_hbm, idx_v)
        lanes = lax.iota(jnp.int32, 16)

        def src_chan(c):
            oc = (base + c) % _DIM_LEN
            aligned = pl.multiple_of((oc // 16) * 16, 16)
            vec = idx_v[pl.ds(aligned, 16)]
            return jnp.max(jnp.where(lanes == oc - aligned, vec, 0))

        chans = [None] * _CPW

        def start_gather(c):
            b = (base + c) // _DIM_LEN
            chans[c] = src_chan(c)
            pltpu.async_copy(x_hbm.at[b, chans[c]], bufs[c % 2], gsem)

        def wait_gather(c):
            b = (base + c) // _DIM_LEN
            pltpu.make_async_copy(x_hbm.at[b, chans[c]], bufs[c % 2], gsem).wait()

        def start_write(c):
            b = (base + c) // _DIM_LEN
            pltpu.async_copy(
                bufs[c % 2], out_hbm.at[b, (base + c) % _DIM_LEN], wsem
            )

        def wait_write(c):
            b = (base + c) // _DIM_LEN
            pltpu.make_async_copy(
                bufs[c % 2], out_hbm.at[b, (base + c) % _DIM_LEN], wsem
            ).wait()

        for c in range(_CPW):
            if c >= 2:
                wait_write(c - 2)
            start_gather(c)
            if c >= 1:
                wait_gather(c - 1)
                start_write(c - 1)
        wait_gather(_CPW - 1)
        start_write(_CPW - 1)
        wait_write(_CPW - 2)
        wait_write(_CPW - 1)

    return k(x, src)


def kernel(x):
    src = jnp.asarray(_SRC_CHANNELS)
    return _permute_channels(x, src)
